# baseline (speedup 1.0000x reference)
import functools

import jax
import jax.numpy as jnp
from jax import lax
from jax.experimental import pallas as pl
from jax.experimental.pallas import tpu as pltpu, tpu_sc as plsc

_BS, _L, _DIM = 4, 256, 128
_G = 2
_VAR = _DIM // _G
_C = 512
_EPS = 1e-5
_N = _BS * _L
_HI = jax.lax.Precision.HIGHEST
_NIDX = 4 * _N


def _vq_tc_kernel(x_ref, w0_ref, w1_ref, et_ref, gnw_ref, gnb_ref,
                  z_ref, idx_ref, gap_ref):
    xx = x_ref[...]
    et = et_ref[...]
    gnw = gnw_ref[...]
    gnb = gnb_ref[...]

    en = jnp.sum(et * et, axis=0, keepdims=True)
    lane_c = jax.lax.broadcasted_iota(jnp.int32, (_N, _C), 1)
    inv_cnt = 1.0 / float(_L * _VAR)
    big = jnp.float32(3.4e38)

    for g, w_ref in ((0, w0_ref), (1, w1_ref)):
        cols = slice(g * _VAR, (g + 1) * _VAR)
        xg = xx[:, cols]
        wt = w_ref[...]
        y = jnp.dot(xg, wt, preferred_element_type=jnp.float32)

        y3 = y.reshape(_BS, _L, _VAR)
        bmean = jnp.sum(y3, axis=(1, 2), keepdims=True) * inv_cnt
        dcen3 = y3 - bmean
        bvar = jnp.sum(dcen3 * dcen3, axis=(1, 2), keepdims=True) * inv_cnt
        z3 = dcen3 / jnp.sqrt(bvar + _EPS)
        z = z3.reshape(_N, _VAR)
        z = z * gnw[:, cols] + gnb[:, cols]

        s = en - 2.0 * jnp.dot(z, et, preferred_element_type=jnp.float32,
                               precision=_HI)

        m1 = jnp.min(s, axis=1, keepdims=True)
        i1 = jnp.min(jnp.where(s == m1, lane_c, _C), axis=1, keepdims=True)
        smask = jnp.where(lane_c == i1, big, s)
        m2 = jnp.min(smask, axis=1, keepdims=True)
        i2 = jnp.min(jnp.where(smask == m2, lane_c, _C), axis=1, keepdims=True)

        swap = i2 < i1
        ilo = jnp.where(swap, i2, i1)
        ihi = jnp.where(swap, i1, i2)
        z_ref[:, cols] = z
        idx_ref[:, 2 * g:2 * g + 1] = ilo
        idx_ref[:, 2 * g + 1:2 * g + 2] = ihi
        gap_ref[:, g:g + 1] = m2 - m1


_info = plsc.get_sparse_core_info()
_NW = _info.num_cores * _info.num_subcores
_BPW = _NIDX // _NW


def _make_sc_gather():
    mesh = plsc.VectorSubcoreMesh(core_axis_name="c", subcore_axis_name="s")

    @functools.partial(
        pl.kernel, mesh=mesh,
        out_type=jax.ShapeDtypeStruct((_NIDX, 2 * _VAR), jnp.float32),
        scratch_types=[
            pltpu.VMEM((_BPW,), jnp.int32),
            pltpu.VMEM((_BPW, 2 * _VAR), jnp.float32),
            pltpu.SemaphoreType.DMA,
        ],
    )
    def k(table_hbm, idx_hbm, out_hbm, idx_v, rows_v, sem):
        wid = lax.axis_index("s") * _info.num_cores + lax.axis_index("c")
        base = wid * _BPW
        pltpu.sync_copy(idx_hbm.at[pl.ds(base, _BPW)], idx_v)
        pltpu.async_copy(table_hbm.at[idx_v], rows_v, sem).wait()
        pltpu.sync_copy(rows_v, out_hbm.at[pl.ds(base, _BPW)])

    return k


def kernel(x, conv_w, gn_w, gn_b, emb):
    bs, l, d = x.shape
    x2 = x.reshape(bs * l, d)
    w = conv_w[:, :, 0]
    w0t = w[:_VAR, :].T
    w1t = w[_VAR:, :].T
    e = emb[:, 0, :]
    et = e.T
    gnw2 = gn_w.reshape(1, d)
    gnb2 = gn_b.reshape(1, d)

    z2, idx, gap = pl.pallas_call(
        _vq_tc_kernel,
        out_shape=(jax.ShapeDtypeStruct((bs * l, d), jnp.float32),
                   jax.ShapeDtypeStruct((bs * l, 4), jnp.int32),
                   jax.ShapeDtypeStruct((bs * l, _G), jnp.float32)),
    )(x2, w0t, w1t, et, gnw2, gnb2)

    idx_flat = idx.T.reshape(_NIDX)
    e_pad = jnp.concatenate([e, jnp.zeros_like(e)], axis=1)
    rows = _make_sc_gather()(e_pad, idx_flat)[:, :_VAR]

    ze4 = z2.reshape(bs, l, _G, _VAR)
    lo = jnp.stack([rows[0:_N], rows[2 * _N:3 * _N]], axis=1)
    hi = jnp.stack([rows[_N:2 * _N], rows[3 * _N:4 * _N]], axis=1)
    lo4 = lo.reshape(bs, l, _G, _VAR)
    hi4 = hi.reshape(bs, l, _G, _VAR)
    cand = jnp.stack([lo4, hi4])
    dcand = jnp.linalg.norm(ze4[None] - cand, axis=-1)
    pick_lo = (dcand[0] <= dcand[1])[..., None]
    zq = jnp.where(pick_lo, cand[0], cand[1])

    zf = z2.reshape(_N * _G, _VAR)
    zq_flat = zq.reshape(_N * _G, _VAR)
    flat_gap = gap.reshape(_N * _G)
    ridx = jnp.sort(jax.lax.top_k(-flat_gap, 256)[1])
    zr = jax.lax.optimization_barrier(zf[ridx])
    eb = jax.lax.optimization_barrier(e)
    ndist = jnp.linalg.norm(zr[None] - eb[:, None], axis=-1)
    am = jnp.argmin(ndist, axis=0)
    zq_flat = zq_flat.at[ridx].set(e[am])
    return zq_flat.reshape(bs, l, d)
